# baseline (speedup 1.0000x reference)
import functools

import jax
import jax.numpy as jnp
from jax import lax
from jax.experimental import pallas as pl
from jax.experimental.pallas import tpu as pltpu
from jax.experimental.pallas import tpu_sc as plsc

D = 768
LANES = 16
DV = D // LANES
NC, NS = 2, 16
NW = NC * NS
CHUNK = 32
TGRP = 4
EPS = 1e-12


def _ptsum_body(pos_ref, type_ref, out_ref):
    out_ref[...] = pos_ref[...][:, None, :] + type_ref[...][None, :, :]


def _rsqrt(x):
    bits = plsc.bitcast(x, jnp.int32)
    bits = jnp.int32(0x5F3759DF) - lax.shift_right_logical(
        bits, jnp.full((LANES,), 1, jnp.int32))
    y = plsc.bitcast(bits, jnp.float32)
    for _ in range(3):
        y = y * (1.5 - 0.5 * x * y * y)
    return y


def _make_sc_kernel(n_tok):
    tpw = n_tok // NW
    nch = tpw // CHUNK
    mesh = plsc.VectorSubcoreMesh(
        core_axis_name="c", subcore_axis_name="s",
        num_cores=NC, num_subcores=NS)

    @functools.partial(
        pl.kernel,
        out_type=jax.ShapeDtypeStruct((n_tok, D), jnp.float32),
        mesh=mesh,
        compiler_params=pltpu.CompilerParams(needs_layout_passes=False),
        scratch_types=[
            pltpu.VMEM((tpw,), jnp.int32),
            pltpu.VMEM((tpw,), jnp.int32),
            pltpu.VMEM((CHUNK, D), jnp.float32),
            pltpu.VMEM((CHUNK, D), jnp.float32),
            pltpu.VMEM((CHUNK, D), jnp.float32),
            pltpu.VMEM((CHUNK, D), jnp.float32),
            pltpu.VMEM((D,), jnp.float32),
            pltpu.VMEM((D,), jnp.float32),
            pltpu.SemaphoreType.DMA,
            pltpu.SemaphoreType.DMA,
            pltpu.SemaphoreType.DMA,
            pltpu.SemaphoreType.DMA,
        ],
    )
    def sc_kernel(ids_hbm, gidx_hbm, wtab_hbm, ptab_hbm, gam_hbm, bet_hbm,
                  out_hbm, ids_v, gidx_v, wbuf0, wbuf1, pbuf0, pbuf1,
                  gam_v, bet_v, semw0, semw1, semp0, semp1):
        wid = lax.axis_index("s") * NC + lax.axis_index("c")
        base0 = wid * tpw
        pltpu.sync_copy(ids_hbm.at[pl.ds(base0, tpw)], ids_v)
        pltpu.sync_copy(gidx_hbm.at[pl.ds(base0, tpw)], gidx_v)
        pltpu.sync_copy(gam_hbm, gam_v)
        pltpu.sync_copy(bet_hbm, bet_v)

        wbufs = (wbuf0, wbuf1)
        pbufs = (pbuf0, pbuf1)
        semws = (semw0, semw1)
        semps = (semp0, semp1)

        def issue(chunk, b):
            off = chunk * CHUNK
            pltpu.async_copy(wtab_hbm.at[ids_v.at[pl.ds(off, CHUNK)]],
                             wbufs[b], semws[b])
            pltpu.async_copy(ptab_hbm.at[gidx_v.at[pl.ds(off, CHUNK)]],
                             pbufs[b], semps[b])

        def drain(b):
            pltpu.make_async_copy(wtab_hbm.at[ids_v.at[pl.ds(0, CHUNK)]],
                                  wbufs[b], semws[b]).wait()
            pltpu.make_async_copy(ptab_hbm.at[gidx_v.at[pl.ds(0, CHUNK)]],
                                  pbufs[b], semps[b]).wait()

        issue(0, 0)

        def compute(chunk, b):
            wbuf, pbuf = wbufs[b], pbufs[b]

            def grp_body(g, carry):
                t0 = g * TGRP
                accs = [[jnp.zeros((LANES,), jnp.float32) for _ in range(2)]
                        for _ in range(TGRP)]
                acc2s = [[jnp.zeros((LANES,), jnp.float32) for _ in range(2)]
                         for _ in range(TGRP)]
                for d in range(DV):
                    sl = pl.ds(d * LANES, LANES)
                    for k in range(TGRP):
                        e = wbuf[t0 + k, sl] + pbuf[t0 + k, sl]
                        wbuf[t0 + k, sl] = e
                        accs[k][d % 2] = accs[k][d % 2] + e
                        acc2s[k][d % 2] = acc2s[k][d % 2] + e * e
                means, rstds = [], []
                for k in range(TGRP):
                    s1 = jnp.broadcast_to(
                        jnp.sum(accs[k][0] + accs[k][1]), (LANES,))
                    s2 = jnp.broadcast_to(
                        jnp.sum(acc2s[k][0] + acc2s[k][1]), (LANES,))
                    meanv = s1 * (1.0 / D)
                    varv = s2 * (1.0 / D) - meanv * meanv
                    means.append(meanv)
                    rstds.append(_rsqrt(varv + EPS))
                for d in range(DV):
                    sl = pl.ds(d * LANES, LANES)
                    gm = gam_v[sl]
                    bt = bet_v[sl]
                    for k in range(TGRP):
                        e = wbuf[t0 + k, sl]
                        wbuf[t0 + k, sl] = (e - means[k]) * rstds[k] * gm + bt
                return carry

            lax.fori_loop(0, CHUNK // TGRP, grp_body, 0)
            pltpu.sync_copy(wbuf, out_hbm.at[pl.ds(base0 + chunk * CHUNK,
                                                   CHUNK)])

        def chunk_body(ci, carry):
            for b in range(2):
                chunk = ci * 2 + b
                @pl.when(chunk + 1 < nch)
                def _():
                    issue(chunk + 1, 1 - b)
                drain(b)
                compute(chunk, b)
            return carry

        lax.fori_loop(0, nch // 2, chunk_body, 0)

    return sc_kernel


def kernel(input_ids, token_type_ids, word_emb, pos_emb, type_emb,
           ln_gamma, ln_beta):
    B, S = input_ids.shape
    n_tok = B * S
    n_types = type_emb.shape[0]
    ids = input_ids.reshape(-1).astype(jnp.int32)
    gidx = (n_types * jnp.arange(S, dtype=jnp.int32)[None, :]
            + token_type_ids.astype(jnp.int32)).reshape(-1)
    ptab = pl.pallas_call(
        _ptsum_body,
        out_shape=jax.ShapeDtypeStruct(
            (pos_emb.shape[0], n_types, D), jnp.float32),
    )(pos_emb, type_emb).reshape(-1, D)
    out = _make_sc_kernel(n_tok)(ids, gidx, word_emb, ptab, ln_gamma, ln_beta)
    return out.reshape(B, S, D)


# speedup vs baseline: 1.3011x; 1.3011x over previous
import functools

import jax
import jax.numpy as jnp
from jax import lax
from jax.experimental import pallas as pl
from jax.experimental.pallas import tpu as pltpu
from jax.experimental.pallas import tpu_sc as plsc

D = 768
LANES = 16
DV = D // LANES
NC, NS = 2, 16
NW = NC * NS
CHUNK = 32
TGRP = 4
EPS = 1e-12


def _ptsum_body(pos_ref, type_ref, out_ref):
    out_ref[...] = pos_ref[...][:, None, :] + type_ref[...][None, :, :]


def _rsqrt(x):
    bits = plsc.bitcast(x, jnp.int32)
    bits = jnp.int32(0x5F3759DF) - lax.shift_right_logical(
        bits, jnp.full((LANES,), 1, jnp.int32))
    y = plsc.bitcast(bits, jnp.float32)
    for _ in range(3):
        y = y * (1.5 - 0.5 * x * y * y)
    return y


def _make_sc_kernel(n_tok):
    tpw = n_tok // NW
    nch = tpw // CHUNK
    mesh = plsc.VectorSubcoreMesh(
        core_axis_name="c", subcore_axis_name="s",
        num_cores=NC, num_subcores=NS)

    @functools.partial(
        pl.kernel,
        out_type=jax.ShapeDtypeStruct((n_tok, D), jnp.float32),
        mesh=mesh,
        compiler_params=pltpu.CompilerParams(needs_layout_passes=False),
        scratch_types=[
            pltpu.VMEM((tpw,), jnp.int32),
            pltpu.VMEM((tpw,), jnp.int32),
            pltpu.VMEM((CHUNK, D), jnp.float32),
            pltpu.VMEM((CHUNK, D), jnp.float32),
            pltpu.VMEM((CHUNK, D), jnp.float32),
            pltpu.VMEM((CHUNK, D), jnp.float32),
            pltpu.VMEM((D,), jnp.float32),
            pltpu.VMEM((D,), jnp.float32),
            pltpu.SemaphoreType.DMA,
            pltpu.SemaphoreType.DMA,
            pltpu.SemaphoreType.DMA,
            pltpu.SemaphoreType.DMA,
        ],
    )
    def sc_kernel(ids_hbm, gidx_hbm, wtab_hbm, ptab_hbm, gam_hbm, bet_hbm,
                  out_hbm, ids_v, gidx_v, wbuf0, wbuf1, pbuf0, pbuf1,
                  gam_v, bet_v, semw0, semw1, semp0, semp1):
        wid = lax.axis_index("s") * NC + lax.axis_index("c")
        base0 = wid * tpw
        pltpu.sync_copy(ids_hbm.at[pl.ds(base0, tpw)], ids_v)
        pltpu.sync_copy(gidx_hbm.at[pl.ds(base0, tpw)], gidx_v)
        pltpu.sync_copy(gam_hbm, gam_v)
        pltpu.sync_copy(bet_hbm, bet_v)

        wbufs = (wbuf0, wbuf1)
        pbufs = (pbuf0, pbuf1)
        semws = (semw0, semw1)
        semps = (semp0, semp1)

        def issue(chunk, b):
            off = chunk * CHUNK
            pltpu.async_copy(wtab_hbm.at[ids_v.at[pl.ds(off, CHUNK)]],
                             wbufs[b], semws[b])
            pltpu.async_copy(ptab_hbm.at[gidx_v.at[pl.ds(off, CHUNK)]],
                             pbufs[b], semps[b])

        def drain(b):
            pltpu.make_async_copy(wtab_hbm.at[ids_v.at[pl.ds(0, CHUNK)]],
                                  wbufs[b], semws[b]).wait()
            pltpu.make_async_copy(ptab_hbm.at[gidx_v.at[pl.ds(0, CHUNK)]],
                                  pbufs[b], semps[b]).wait()

        issue(0, 0)

        def compute(chunk, b):
            wbuf, pbuf = wbufs[b], pbufs[b]

            @plsc.parallel_loop(0, CHUNK, unroll=TGRP)
            def tok_body(t):
                accs = [jnp.zeros((LANES,), jnp.float32) for _ in range(2)]
                acc2s = [jnp.zeros((LANES,), jnp.float32) for _ in range(2)]
                for d in range(DV):
                    sl = pl.ds(d * LANES, LANES)
                    e = wbuf[t, sl] + pbuf[t, sl]
                    wbuf[t, sl] = e
                    accs[d % 2] = accs[d % 2] + e
                    acc2s[d % 2] = acc2s[d % 2] + e * e
                s1 = jnp.broadcast_to(jnp.sum(accs[0] + accs[1]), (LANES,))
                s2 = jnp.broadcast_to(jnp.sum(acc2s[0] + acc2s[1]), (LANES,))
                meanv = s1 * (1.0 / D)
                varv = s2 * (1.0 / D) - meanv * meanv
                rstdv = _rsqrt(varv + EPS)
                for d in range(DV):
                    sl = pl.ds(d * LANES, LANES)
                    e = wbuf[t, sl]
                    wbuf[t, sl] = (e - meanv) * rstdv * gam_v[sl] + bet_v[sl]

            pltpu.sync_copy(wbuf, out_hbm.at[pl.ds(base0 + chunk * CHUNK,
                                                   CHUNK)])

        def chunk_body(ci, carry):
            for b in range(2):
                chunk = ci * 2 + b
                @pl.when(chunk + 1 < nch)
                def _():
                    issue(chunk + 1, 1 - b)
                drain(b)
                compute(chunk, b)
            return carry

        lax.fori_loop(0, nch // 2, chunk_body, 0)

    return sc_kernel


def kernel(input_ids, token_type_ids, word_emb, pos_emb, type_emb,
           ln_gamma, ln_beta):
    B, S = input_ids.shape
    n_tok = B * S
    n_types = type_emb.shape[0]
    ids = input_ids.reshape(-1).astype(jnp.int32)
    gidx = (n_types * jnp.arange(S, dtype=jnp.int32)[None, :]
            + token_type_ids.astype(jnp.int32)).reshape(-1)
    ptab = pl.pallas_call(
        _ptsum_body,
        out_shape=jax.ShapeDtypeStruct(
            (pos_emb.shape[0], n_types, D), jnp.float32),
    )(pos_emb, type_emb).reshape(-1, D)
    out = _make_sc_kernel(n_tok)(ids, gidx, word_emb, ptab, ln_gamma, ln_beta)
    return out.reshape(B, S, D)
